# baseline (speedup 1.0000x reference)
import functools

import jax
import jax.numpy as jnp
from jax.experimental import pallas as pl
from jax.experimental.pallas import tpu as pltpu

N_GENES = 2000
FILTER_G = [4, 64, 32]
FILTER_D = [128, 64, 32]
N_TS = FILTER_G[0]
LOSS_T_WEIGHT = 0.5


def _ta_kernel(wc_ref, bc_ref, temp_ref, gamma_ref, beta_ref, out_ref):
    y = jnp.dot(wc_ref[...], temp_ref[...], preferred_element_type=jnp.float32)
    y = y + bc_ref[...]
    mean = jnp.mean(y, axis=0, keepdims=True)
    var = jnp.mean((y - mean) ** 2, axis=0, keepdims=True)
    y = (y - mean) * jax.lax.rsqrt(var + 1e-05) * gamma_ref[...] + beta_ref[...]
    out_ref[...] = jax.nn.sigmoid(y)


def _ta_block(temp, p):
    f = temp.shape[1]
    return pl.pallas_call(
        _ta_kernel,
        out_shape=jax.ShapeDtypeStruct((N_GENES, f), jnp.float32),
    )(p['Wc'], p['bc'][:, None], temp, p['gamma'][None, :], p['beta'][None, :])


def _gcn_conv(x, src, dst, ew, W, b):
    n = x.shape[0]
    loop = jnp.arange(n, dtype=src.dtype)
    src2 = jnp.concatenate([src, loop])
    dst2 = jnp.concatenate([dst, loop])
    ew2 = jnp.concatenate([ew, jnp.ones((n,), x.dtype)])
    deg = jax.ops.segment_sum(ew2, dst2, num_segments=n)
    dinv = jnp.where(deg > 0, 1.0 / jnp.sqrt(deg), 0.0)
    norm = dinv[src2] * ew2 * dinv[dst2]
    h = x @ W.T
    out = jax.ops.segment_sum(norm[:, None] * h[src2], dst2, num_segments=n)
    return out + b


def _gat_conv(x, src, dst, p):
    n = x.shape[0]
    loop = jnp.arange(n, dtype=src.dtype)
    src2 = jnp.concatenate([src, loop])
    dst2 = jnp.concatenate([dst, loop])
    h = x @ p['W'].T
    a_src = h @ p['att_src']
    a_dst = h @ p['att_dst']
    e = a_src[src2] + a_dst[dst2]
    e = jnp.where(e > 0, e, 0.2 * e)
    emax = jax.ops.segment_max(e, dst2, num_segments=n)
    emax = jnp.where(jnp.isfinite(emax), emax, 0.0)
    ex = jnp.exp(e - emax[dst2])
    den = jax.ops.segment_sum(ex, dst2, num_segments=n)
    coef = ex / den[dst2]
    out = jax.ops.segment_sum(coef[:, None] * h[src2], dst2, num_segments=n)
    return out + p['b']


def _kl_batchmean(inp, target):
    pw = jnp.where(target > 0, target * jnp.log(target), 0.0) - target * inp
    return pw.sum() / inp.shape[0]


def kernel(x_g, edges_g, weights, x_d, edges_d, params):
    ng = len(params['gcn'])
    h = x_g
    for i in range(ng):
        p = params['gcn'][i]
        h = _gcn_conv(h, edges_g[0], edges_g[1], weights, p['W'], p['b'])
        h = jax.nn.sigmoid(h) if i == ng - 1 else jax.nn.relu(h)
    x_g_enc = h
    g = x_g_enc
    nd = len(params['dec_g'])
    for i in range(nd):
        p = params['dec_g'][i]
        g = g @ p['W'].T + p['b']
        g = jax.nn.sigmoid(g) if i == nd - 1 else jax.nn.relu(g)
    loss_g = jnp.mean((x_g - g) ** 2) + _kl_batchmean(x_g, g)
    nl = len(FILTER_D) - 1
    xs = [x_d[j] for j in range(N_TS - 1)]
    temp = xs[0]
    for i in range(nl):
        new = []
        for j in range(N_TS - 1):
            if i == 0 or j == 0:
                temp = xs[j]
                out = _gat_conv(xs[j], edges_d[j, 0], edges_d[j, 1], params['gat'][i][j])
            else:
                mask = _ta_block(temp, params['ta'][i - 1][j - 1])
                temp = xs[j] * mask
                out = _gat_conv(temp, edges_d[j, 0], edges_d[j, 1], params['gat'][i][j])
            out = jax.nn.sigmoid(out) if i == nl - 1 else jax.nn.relu(out)
            new.append(out)
        xs = new
    final = [xs[0]]
    temp = xs[0]
    for j in range(N_TS - 2):
        mask = _ta_block(temp, params['ta'][nl - 1][j])
        temp = xs[j + 1] * mask
        final.append(temp)
    x_d_enc = jnp.stack(final)
    ds = [x_d_enc[j] for j in range(N_TS - 1)]
    nld = len(params['dec_d'])
    for i in range(nld):
        new = []
        for j in range(N_TS - 1):
            p = params['dec_d'][i][j]
            o = ds[j] @ p['W'].T + p['b']
            o = jax.nn.sigmoid(o) if i == nld - 1 else jax.nn.relu(o)
            new.append(o)
        ds = new
    dx = jnp.stack(ds)
    loss_d = jnp.mean((x_d[N_TS - 2] - dx[N_TS - 2]) ** 2) + _kl_batchmean(x_d[N_TS - 2], dx[N_TS - 2])
    w = 1.0 / (N_TS - 1)
    loss_t = 0.0
    for i in range(N_TS - 2):
        loss_t = loss_t + w * _kl_batchmean(dx[i], dx[i + 1])
    x_encoded = jnp.concatenate([x_g_enc, x_d_enc[N_TS - 2]], axis=1)
    total_loss = loss_g + loss_d + LOSS_T_WEIGHT * loss_t
    return (x_encoded, total_loss)


# speedup vs baseline: 30.0220x; 30.0220x over previous
import functools

import jax
import jax.numpy as jnp
from jax import lax
from jax.experimental import pallas as pl
from jax.experimental.pallas import tpu as pltpu
from jax.experimental.pallas import tpu_sc as plsc

N = 2000
E = 64000
FILTER_G = [4, 64, 32]
FILTER_D = [128, 64, 32]
N_TS = FILTER_G[0]
LOSS_T_WEIGHT = 0.5

NC, NS, L = 2, 16, 16
NW = NC * NS
ECHT = E // NS
NCG = 4
ZMAX = 4000

_MESH = plsc.VectorSubcoreMesh(core_axis_name="c", subcore_axis_name="s")
_F32 = jnp.float32


def _sc_edge_body(specs, *refs):
    n_in = {"deg": 2, "gcn": 5, "gat": 6}
    n_out = {"deg": 1, "gcn": 1, "gat": 2}
    ins, outs = [], []
    pos = 0
    for sp in specs:
        ins.append(refs[pos:pos + n_in[sp[0]]])
        pos += n_in[sp[0]]
    zeros_hbm = refs[pos]
    pos += 1
    for sp in specs:
        outs.append(refs[pos:pos + n_out[sp[0]]])
        pos += n_out[sp[0]]
    (src_v, dst_v, ex_v, asrc_v, adst_v, col_v, c_v, ew_v, dinv_v,
     colg_v, colg2_v, vals_v, vals2_v, idx_v, zbuf_v, acc_sh, den_sh,
     sem_c0, sem_c1, sem_s0, sem_s1, sem_z) = refs[pos:]

    cid = lax.axis_index("c")
    sid = lax.axis_index("s")
    ebase = sid * ECHT

    pltpu.sync_copy(zeros_hbm.at[pl.ds(0, ZMAX)], zbuf_v)

    def zero_acc_start(F):
        sl = (F // 2) * N // NS
        return pltpu.async_copy(zbuf_v.at[pl.ds(0, sl)],
                                acc_sh.at[pl.ds(sid * sl, sl)], sem_z)

    def zero_den():
        pltpu.sync_copy(zbuf_v.at[pl.ds(0, N)], den_sh)

    def den_out(dout):
        pltpu.sync_copy(den_sh, col_v)
        pltpu.sync_copy(col_v, dout)

    def row_pass(hT, F, accp):
        FH = F // 2
        ngrp = FH // NCG
        colbase = cid * FH

        @plsc.parallel_loop(0, ECHT // L, unroll=4)
        def _(k):
            sl = pl.ds(k * L, L)
            i_d = dst_v[sl]
            for ci in range(NCG):
                idx_v[pl.ds(ci * ECHT + k * L, L)] = i_d + ci * N

        cols = (colg_v, colg2_v)
        vals = (vals_v, vals2_v)
        csem = (sem_c0, sem_c1)
        ssem = (sem_s0, sem_s1)
        cdesc = [None, None]
        sdesc = [None, None]
        cdesc[0] = pltpu.async_copy(hT.at[pl.ds(colbase * N, NCG * N)],
                                    cols[0], csem[0])
        for g in range(ngrp):
            b = g % 2
            nb = (g + 1) % 2
            if g + 1 < ngrp:
                cdesc[nb] = pltpu.async_copy(
                    hT.at[pl.ds((colbase + (g + 1) * NCG) * N, NCG * N)],
                    cols[nb], csem[nb])
            cdesc[b].wait()
            if sdesc[b] is not None:
                sdesc[b].wait()

            @plsc.parallel_loop(0, ECHT // L, unroll=4)
            def _(k, b=b):
                sl = pl.ds(k * L, L)
                i_s = src_v[sl]
                exv = ex_v[sl]
                for ci in range(NCG):
                    vals[b][pl.ds(ci * ECHT + k * L, L)] = (
                        plsc.load_gather(cols[b], [i_s + ci * N]) * exv)

            sdesc[b] = pltpu.async_copy(
                vals[b], acc_sh.at[pl.ds(g * NCG * N, NCG * N)].at[idx_v],
                ssem[b], add=True)
        for d in sdesc:
            if d is not None:
                d.wait()
        plsc.subcore_barrier()
        osl = FH * N // NS
        pltpu.sync_copy(acc_sh.at[pl.ds(sid * osl, osl)],
                        vals_v.at[pl.ds(0, osl)])
        pltpu.sync_copy(vals_v.at[pl.ds(0, osl)],
                        accp.at[pl.ds(cid * (FH * N) + sid * osl, osl)])

    for sp, iref, oref in zip(specs, ins, outs):
        kind = sp[0]
        if kind == "deg":
            dst_h, ew_h = iref
            (degp,) = oref

            @pl.when(cid == 0)
            def _():
                pltpu.sync_copy(dst_h.at[pl.ds(ebase, ECHT)], dst_v)
                pltpu.sync_copy(ew_h.at[pl.ds(ebase, ECHT)], ew_v)

                @pl.when(sid == 0)
                def _():
                    zero_den()

                plsc.subcore_barrier()
                pltpu.sync_copy(ew_v, den_sh.at[dst_v], add=True)
                plsc.subcore_barrier()

                @pl.when(sid == 0)
                def _():
                    den_out(degp)

                plsc.subcore_barrier()
        elif kind == "gcn":
            src_h, dst_h, ew_h, dinv_h, hT = iref
            F = sp[1]
            (accp,) = oref
            zdesc = zero_acc_start(F)
            pltpu.sync_copy(src_h.at[pl.ds(ebase, ECHT)], src_v)
            pltpu.sync_copy(dst_h.at[pl.ds(ebase, ECHT)], dst_v)
            pltpu.sync_copy(ew_h.at[pl.ds(ebase, ECHT)], ew_v)
            pltpu.sync_copy(dinv_h, dinv_v)

            @plsc.parallel_loop(0, ECHT // L, unroll=4)
            def _(k):
                sl = pl.ds(k * L, L)
                ex_v[sl] = plsc.load_gather(dinv_v, [src_v[sl]]) * ew_v[sl]

            zdesc.wait()
            plsc.subcore_barrier()
            row_pass(hT, F, accp)
            plsc.subcore_barrier()
        else:
            src_h, dst_h, asrc_h, adst_h, cvec_h, hT = iref
            F = sp[1]
            accp, denp = oref
            zdesc = zero_acc_start(F)
            pltpu.sync_copy(src_h.at[pl.ds(ebase, ECHT)], src_v)
            pltpu.sync_copy(dst_h.at[pl.ds(ebase, ECHT)], dst_v)
            pltpu.sync_copy(asrc_h, asrc_v)
            pltpu.sync_copy(adst_h, adst_v)
            pltpu.sync_copy(cvec_h, c_v)

            @pl.when(jnp.logical_and(cid == 0, sid == 0))
            def _():
                zero_den()

            cshift = c_v[...]

            @plsc.parallel_loop(0, ECHT // L, unroll=4)
            def _(k):
                sl = pl.ds(k * L, L)
                e = (plsc.load_gather(asrc_v, [src_v[sl]])
                     + plsc.load_gather(adst_v, [dst_v[sl]]))
                e = jnp.where(e > 0, e, 0.2 * e)
                ex_v[sl] = jnp.exp(e - cshift)

            zdesc.wait()
            plsc.subcore_barrier()

            @pl.when(cid == 0)
            def _():
                pltpu.sync_copy(ex_v, den_sh.at[dst_v], add=True)

            row_pass(hT, F, accp)

            @pl.when(jnp.logical_and(cid == 0, sid == 0))
            def _():
                den_out(denp)

            plsc.subcore_barrier()


def _sc_edge_launch(specs, fmax, args):
    out_type = []
    for sp in specs:
        if sp[0] == "deg":
            out_type.append(jax.ShapeDtypeStruct((N,), _F32))
        elif sp[0] == "gcn":
            out_type.append(jax.ShapeDtypeStruct((sp[1] * N,), _F32))
        else:
            out_type.append(jax.ShapeDtypeStruct((sp[1] * N,), _F32))
            out_type.append(jax.ShapeDtypeStruct((N,), _F32))
    scratch = [
        pltpu.VMEM((ECHT,), jnp.int32),
        pltpu.VMEM((ECHT,), jnp.int32),
        pltpu.VMEM((ECHT,), _F32),
        pltpu.VMEM((N,), _F32),
        pltpu.VMEM((N,), _F32),
        pltpu.VMEM((N,), _F32),
        pltpu.VMEM((L,), _F32),
        pltpu.VMEM((ECHT,), _F32),
        pltpu.VMEM((N,), _F32),
        pltpu.VMEM((NCG * N,), _F32),
        pltpu.VMEM((NCG * N,), _F32),
        pltpu.VMEM((NCG * ECHT,), _F32),
        pltpu.VMEM((NCG * ECHT,), _F32),
        pltpu.VMEM((NCG * ECHT,), jnp.int32),
        pltpu.VMEM((ZMAX,), _F32),
        pltpu.VMEM_SHARED((fmax // 2 * N,), _F32),
        pltpu.VMEM_SHARED((N,), _F32),
        pltpu.SemaphoreType.DMA,
        pltpu.SemaphoreType.DMA,
        pltpu.SemaphoreType.DMA,
        pltpu.SemaphoreType.DMA,
        pltpu.SemaphoreType.DMA,
    ]
    kfn = pl.kernel(
        functools.partial(_sc_edge_body, tuple(specs)),
        out_type=tuple(out_type),
        mesh=_MESH,
        scratch_types=scratch,
        compiler_params=pltpu.CompilerParams(needs_layout_passes=False),
    )
    return kfn(*args)


def _pc(body, out_shapes):
    return pl.pallas_call(body, out_shape=out_shapes)


def _dotg(a, b, ca, cb):
    return lax.dot_general(a, b, (((ca,), (cb,)), ((), ())),
                           preferred_element_type=_F32)


def _leaky(x):
    return jnp.where(x > 0, x, 0.2 * x)


def _gat_fin(accp, denp, a_s, a_d, c, ht, b, last):
    es = jnp.exp(_leaky(a_s + a_d) - c)
    den = denp + es
    acc = accp + es * ht
    o = acc / den + b
    return jax.nn.sigmoid(o) if last else jnp.maximum(o, 0.0)


def _ta_mul(wc, bc, tempt, gamma, beta, xint):
    y = _dotg(tempt, wc, 1, 1) + bc
    mean = jnp.mean(y, axis=1, keepdims=True)
    var = jnp.mean((y - mean) ** 2, axis=1, keepdims=True)
    y = (y - mean) * lax.rsqrt(var + 1e-05) * gamma + beta
    return xint * jax.nn.sigmoid(y)


def _kl_t(inp_t, target_t):
    pw = jnp.where(target_t > 0, target_t * jnp.log(target_t), 0.0) \
        - target_t * inp_t
    return jnp.sum(pw) * (1.0 / N)


def _t0_body(xg_ref, wg1_ref, xd0, w0, as0, ad0, xd1, w1, as1, ad1,
             xd2, w2, as2, ad2,
             h1t_o, ht0_o, as0_o, ad0_o, c0_o, ht1_o, as1_o, ad1_o, c1_o,
             ht2_o, as2_o, ad2_o, c2_o):
    h1t_o[...] = _dotg(wg1_ref[...], xg_ref[...], 1, 1)
    for x, w, a_s, a_d, ho, aso, ado, co in (
            (xd0, w0, as0, ad0, ht0_o, as0_o, ad0_o, c0_o),
            (xd1, w1, as1, ad1, ht1_o, as1_o, ad1_o, c1_o),
            (xd2, w2, as2, ad2, ht2_o, as2_o, ad2_o, c2_o)):
        ht = _dotg(w[...], x[...], 1, 1)
        av = _dotg(a_s[...], ht, 1, 0)
        bv = _dotg(a_d[...], ht, 1, 0)
        ho[...] = ht
        aso[...] = av
        ado[...] = bv
        co[...] = _leaky(jnp.max(av) + jnp.max(bv))[None, None]


def _t1_body(degp_ref,
             acc0, den0, as0, ad0, c0, ht0, b0,
             acc1, den1, as1, ad1, c1, ht1, b1,
             acc2, den2, as2, ad2, c2, ht2, b2,
             wc00, bc00, g00, be00, wc01, bc01, g01, be01,
             w10, s10, d10, w11, s11, d11, w12, s12, d12,
             dinv_o, ht0_o, as0_o, ad0_o, c0_o, ht1_o, as1_o, ad1_o, c1_o,
             ht2_o, as2_o, ad2_o, c2_o):
    dinv_o[...] = lax.rsqrt(degp_ref[...] + 1.0)
    xs0 = _gat_fin(acc0[...], den0[...], as0[...], ad0[...], c0[...],
                   ht0[...], b0[...], False)
    xs1 = _gat_fin(acc1[...], den1[...], as1[...], ad1[...], c1[...],
                   ht1[...], b1[...], False)
    xs2 = _gat_fin(acc2[...], den2[...], as2[...], ad2[...], c2[...],
                   ht2[...], b2[...], False)
    t1 = _ta_mul(wc00[...], bc00[...], xs0, g00[...], be00[...], xs1)
    t2 = _ta_mul(wc01[...], bc01[...], t1, g01[...], be01[...], xs2)
    for x, w, a_s, a_d, ho, aso, ado, co in (
            (xs0, w10, s10, d10, ht0_o, as0_o, ad0_o, c0_o),
            (t1, w11, s11, d11, ht1_o, as1_o, ad1_o, c1_o),
            (t2, w12, s12, d12, ht2_o, as2_o, ad2_o, c2_o)):
        ht = _dotg(w[...], x, 1, 0)
        av = _dotg(a_s[...], ht, 1, 0)
        bv = _dotg(a_d[...], ht, 1, 0)
        ho[...] = ht
        aso[...] = av
        ado[...] = bv
        co[...] = _leaky(jnp.max(av) + jnp.max(bv))[None, None]


def _t2a_body(accg_ref, dinv_ref, h1t_ref, bg1_ref, wg2_ref, h2t_o):
    dinv = dinv_ref[...]
    acc = accg_ref[...]
    x2 = jnp.maximum(dinv * acc + dinv * dinv * h1t_ref[...] + bg1_ref[...],
                     0.0)
    h2t_o[...] = _dotg(wg2_ref[...], x2, 1, 0)


def _t2b_body(acc0, den0, as0, ad0, c0, ht0, b0,
              acc1, den1, as1, ad1, c1, ht1, b1,
              acc2, den2, as2, ad2, c2, ht2, b2,
              wc10, bc10, g10, be10, wc11, bc11, g11, be11,
              wd10, bd10, wd20, bd20, wd11, bd11, wd21, bd21,
              wd12, bd12, wd22, bd22, xd2t_ref,
              xdenc2_o, ld_o, lt_o):
    xs0 = _gat_fin(acc0[...], den0[...], as0[...], ad0[...], c0[...],
                   ht0[...], b0[...], True)
    xs1 = _gat_fin(acc1[...], den1[...], as1[...], ad1[...], c1[...],
                   ht1[...], b1[...], True)
    xs2 = _gat_fin(acc2[...], den2[...], as2[...], ad2[...], c2[...],
                   ht2[...], b2[...], True)
    ft1 = _ta_mul(wc10[...], bc10[...], xs0, g10[...], be10[...], xs1)
    ft2 = _ta_mul(wc11[...], bc11[...], ft1, g11[...], be11[...], xs2)
    xdenc2_o[...] = ft2

    def dec(dt, w1, b1, w2, b2):
        o = jnp.maximum(_dotg(w1[...], dt, 1, 0) + b1[...], 0.0)
        return jax.nn.sigmoid(_dotg(w2[...], o, 1, 0) + b2[...])

    dx0 = dec(xs0, wd10, bd10, wd20, bd20)
    dx1 = dec(ft1, wd11, bd11, wd21, bd21)
    dx2 = dec(ft2, wd12, bd12, wd22, bd22)
    xd2 = xd2t_ref[...]
    ld_o[...] = (jnp.mean((xd2 - dx2) ** 2) + _kl_t(xd2, dx2))[None, None]
    w = 1.0 / (N_TS - 1)
    lt_o[...] = (w * (_kl_t(dx0, dx1) + _kl_t(dx1, dx2)))[None, None]


def _t3_body(accg_ref, dinv_ref, h2t_ref, bg2_ref,
             wdg1, bdg1, wdg2, bdg2, xg_ref, xdenc2_ref, ld_ref, lt_ref,
             xenc_o, loss_o):
    dinv = dinv_ref[...]
    acc = accg_ref[...]
    xgt = jax.nn.sigmoid(dinv * acc + dinv * dinv * h2t_ref[...]
                         + bg2_ref[...])
    g1 = jnp.maximum(_dotg(wdg1[...], xgt, 1, 0) + bdg1[...], 0.0)
    g = jax.nn.sigmoid(_dotg(wdg2[...], g1, 1, 0) + bdg2[...])
    xgt4 = _dotg(jnp.eye(N_TS, dtype=_F32), xg_ref[...], 1, 1)
    loss_g = jnp.mean((xgt4 - g) ** 2) + _kl_t(xgt4, g)
    xenc_o[...] = jnp.concatenate([xgt, xdenc2_ref[...]], axis=0)
    loss_o[...] = (loss_g + ld_ref[0, 0]
                   + LOSS_T_WEIGHT * lt_ref[0, 0])[None, None]


def kernel(x_g, edges_g, weights, x_d, edges_d, params):
    f1, f2 = FILTER_G[1], FILTER_G[2]
    d1, d2 = FILTER_D[1], FILTER_D[2]
    src_g, dst_g = edges_g[0], edges_g[1]
    zeros = jnp.zeros((max(f1, d1) * N,), _F32)
    gat0, gat1 = params["gat"][0], params["gat"][1]

    t0_out = [jax.ShapeDtypeStruct((f1, N), _F32)]
    for _ in range(3):
        t0_out += [jax.ShapeDtypeStruct((d1, N), _F32),
                   jax.ShapeDtypeStruct((1, N), _F32),
                   jax.ShapeDtypeStruct((1, N), _F32),
                   jax.ShapeDtypeStruct((1, 1), _F32)]
    t0_args = [x_g, params["gcn"][0]["W"]]
    for j in range(3):
        t0_args += [x_d[j], gat0[j]["W"], gat0[j]["att_src"][None, :],
                    gat0[j]["att_dst"][None, :]]
    t0 = _pc(_t0_body, tuple(t0_out))(*t0_args)
    h1t = t0[0]
    prep0 = [t0[1 + 4 * j:5 + 4 * j] for j in range(3)]

    args1 = [dst_g, weights]
    specs1 = [("deg",)]
    for j in range(3):
        ht, asrc, adst, c = prep0[j]
        args1 += [edges_d[j, 0], edges_d[j, 1], asrc.reshape(N),
                  adst.reshape(N), jnp.full((L,), c[0, 0], _F32),
                  ht.reshape(-1)]
        specs1.append(("gat", d1))
    args1.append(zeros)
    out1 = _sc_edge_launch(specs1, d1, args1)
    degp = out1[0]
    gat0p = [(out1[1 + 2 * j], out1[2 + 2 * j]) for j in range(3)]

    t1_out = [jax.ShapeDtypeStruct((1, N), _F32)]
    for _ in range(3):
        t1_out += [jax.ShapeDtypeStruct((d2, N), _F32),
                   jax.ShapeDtypeStruct((1, N), _F32),
                   jax.ShapeDtypeStruct((1, N), _F32),
                   jax.ShapeDtypeStruct((1, 1), _F32)]
    t1_args = [degp.reshape(1, N)]
    for j in range(3):
        ht, asrc, adst, c = prep0[j]
        accp, denp = gat0p[j]
        t1_args += [accp.reshape(d1, N), denp.reshape(1, N),
                    asrc, adst, c, ht, gat0[j]["b"][:, None]]
    for jj in range(2):
        pta = params["ta"][0][jj]
        t1_args += [pta["Wc"], pta["bc"][None, :], pta["gamma"][:, None],
                    pta["beta"][:, None]]
    for j in range(3):
        t1_args += [gat1[j]["W"], gat1[j]["att_src"][None, :],
                    gat1[j]["att_dst"][None, :]]
    t1o = _pc(_t1_body, tuple(t1_out))(*t1_args)
    dinv = t1o[0]
    prep1 = [t1o[1 + 4 * j:5 + 4 * j] for j in range(3)]

    args2 = [src_g, dst_g, weights, dinv.reshape(N), h1t.reshape(-1)]
    specs2 = [("gcn", f1)]
    for j in range(3):
        ht, asrc, adst, c = prep1[j]
        args2 += [edges_d[j, 0], edges_d[j, 1], asrc.reshape(N),
                  adst.reshape(N), jnp.full((L,), c[0, 0], _F32),
                  ht.reshape(-1)]
        specs2.append(("gat", d2))
    args2.append(zeros)
    out2 = _sc_edge_launch(specs2, f1, args2)
    accg1 = out2[0]
    gat1p = [(out2[1 + 2 * j], out2[2 + 2 * j]) for j in range(3)]

    h2t = _pc(_t2a_body, jax.ShapeDtypeStruct((f2, N), _F32))(
        accg1.reshape(f1, N), dinv, h1t, params["gcn"][0]["b"][:, None],
        params["gcn"][1]["W"])

    (accg2,) = _sc_edge_launch(
        [("gcn", f2)], f2, [src_g, dst_g, weights, dinv.reshape(N),
                            h2t.reshape(-1), zeros])

    t2b_args = []
    for j in range(3):
        ht, asrc, adst, c = prep1[j]
        accp, denp = gat1p[j]
        t2b_args += [accp.reshape(d2, N), denp.reshape(1, N),
                     asrc, adst, c, ht, gat1[j]["b"][:, None]]
    for jj in range(2):
        pta = params["ta"][1][jj]
        t2b_args += [pta["Wc"], pta["bc"][None, :], pta["gamma"][:, None],
                     pta["beta"][:, None]]
    pd = params["dec_d"]
    for j in range(3):
        t2b_args += [pd[0][j]["W"], pd[0][j]["b"][:, None],
                     pd[1][j]["W"], pd[1][j]["b"][:, None]]
    t2b_args.append(x_d[N_TS - 2].T)
    xdenc2, loss_d, loss_t = _pc(
        _t2b_body, (jax.ShapeDtypeStruct((d2, N), _F32),
                    jax.ShapeDtypeStruct((1, 1), _F32),
                    jax.ShapeDtypeStruct((1, 1), _F32)))(*t2b_args)

    pg = params["dec_g"]
    xenc_t, loss = _pc(
        _t3_body, (jax.ShapeDtypeStruct((f2 + d2, N), _F32),
                   jax.ShapeDtypeStruct((1, 1), _F32)))(
        accg2.reshape(f2, N), dinv, h2t, params["gcn"][1]["b"][:, None],
        pg[0]["W"], pg[0]["b"][:, None], pg[1]["W"], pg[1]["b"][:, None],
        x_g, xdenc2, loss_d, loss_t)
    return (xenc_t.T, loss[0, 0])
